# baseline (speedup 1.0000x reference)
import functools

import jax
import jax.numpy as jnp
from jax import lax
from jax.experimental import pallas as pl
from jax.experimental.pallas import tpu as pltpu
from jax.experimental.pallas import tpu_sc as plsc

_NC = 2
_NS = 16
_NW = _NC * _NS
_LN_EPS = 1e-5
_CHUNK = 128


def _ln(y, gamma, beta):
    mu = jnp.mean(y, axis=-1, keepdims=True)
    var = jnp.mean((y - mu) ** 2, axis=-1, keepdims=True)
    return (y - mu) * lax.rsqrt(var + _LN_EPS) * gamma + beta


def _silu(x):
    return x * jax.nn.sigmoid(x)


def _enc_body(x_ref, w1_ref, b1_ref, w2_ref, b2_ref, g_ref, be_ref, o_ref):
    h = jnp.dot(x_ref[...], w1_ref[...], preferred_element_type=jnp.float32)
    h = _silu(h + b1_ref[...])
    y = jnp.dot(h, w2_ref[...], preferred_element_type=jnp.float32) + b2_ref[...]
    o_ref[...] = _ln(y, g_ref[...], be_ref[...])


def _edge_body(he_ref, gs_ref, gd_ref, w1_ref, b1_ref, w2_ref, b2_ref, g_ref,
               be_ref, o_ref):
    he = he_ref[...]
    h = jnp.dot(he, w1_ref[...], preferred_element_type=jnp.float32)
    h = _silu(h + (gs_ref[...] + gd_ref[...]) + b1_ref[...])
    y = jnp.dot(h, w2_ref[...], preferred_element_type=jnp.float32) + b2_ref[...]
    o_ref[...] = he + _ln(y, g_ref[...], be_ref[...])


def _node_body(hn_ref, p0_ref, p1_ref, p2_ref, p3_ref, w1a_ref, w1b_ref,
               b1_ref, w2_ref, b2_ref, g_ref, be_ref, o_ref):
    hn = hn_ref[...]
    agg = (p0_ref[...] + p1_ref[...]) + (p2_ref[...] + p3_ref[...])
    h = jnp.dot(hn, w1a_ref[...], preferred_element_type=jnp.float32)
    h = h + jnp.dot(agg, w1b_ref[...], preferred_element_type=jnp.float32)
    h = _silu(h + b1_ref[...])
    y = jnp.dot(h, w2_ref[...], preferred_element_type=jnp.float32) + b2_ref[...]
    o_ref[...] = hn + _ln(y, g_ref[...], be_ref[...])


def _proj_body(hn_ref, ws_ref, wd_ref, ps_ref, pd_ref):
    hn = hn_ref[...]
    ps_ref[...] = jnp.dot(hn, ws_ref[...], preferred_element_type=jnp.float32)
    pd_ref[...] = jnp.dot(hn, wd_ref[...], preferred_element_type=jnp.float32)


def _dec_body(hn_ref, w1_ref, b1_ref, w2_ref, b2_ref, o_ref):
    h = jnp.dot(hn_ref[...], w1_ref[...], preferred_element_type=jnp.float32)
    h = _silu(h + b1_ref[...])
    o_ref[...] = jnp.dot(h, w2_ref[...], preferred_element_type=jnp.float32) + b2_ref[...]


def _row_spec(rb, d):
    return pl.BlockSpec((rb, d), lambda i: (i, 0))


def _full_spec(shape):
    return pl.BlockSpec(shape, lambda i: tuple(0 for _ in shape))


def _call_rows(body, rows, rb, ins, in_dims, out_dims):
    n_row = len(in_dims)
    specs = [_row_spec(rb, d) for d in in_dims]
    specs += [_full_spec(x.shape) for x in ins[n_row:]]
    multi = len(out_dims) > 1
    out_shape = [jax.ShapeDtypeStruct((rows, d), jnp.float32) for d in out_dims]
    out_spec = [_row_spec(rb, d) for d in out_dims]
    if not multi:
        out_shape, out_spec = out_shape[0], out_spec[0]
    return pl.pallas_call(
        body,
        grid=(rows // rb,),
        in_specs=specs,
        out_specs=out_spec,
        out_shape=out_shape,
    )(*ins)


def _sc_mesh():
    return plsc.VectorSubcoreMesh(core_axis_name="c", subcore_axis_name="s",
                                  num_cores=_NC, num_subcores=_NS)


def _make_gather(E, N_pad):
    n_chunks = E // _CHUNK
    n_even = (n_chunks // _NS) * _NS
    per_tile = n_even // _NS
    n_k = per_tile // 3
    rows_per_tile = N_pad // _NS

    @functools.partial(
        pl.kernel,
        out_type=jax.ShapeDtypeStruct((_NC, E, 128), jnp.float32),
        mesh=_sc_mesh(),
        scratch_types=(
            [pltpu.VMEM((_CHUNK,), jnp.int32)] * 3
            + [pltpu.VMEM((_CHUNK, 128), jnp.float32)] * 3
            + [pltpu.VMEM_SHARED((N_pad, 128), jnp.float32)]
            + [pltpu.SemaphoreType.DMA] * 9
        ),
    )
    def gather_k(pt_hbm, sd2_hbm, g2_hbm,
                 ix0, ix1, ix2, bf0, bf1, bf2, table,
                 ls0, ls1, ls2, gs0, gs1, gs2, ws0, ws1, ws2):
        idx_v, buf = [ix0, ix1, ix2], [bf0, bf1, bf2]
        lsem, gsem, wsem = [ls0, ls1, ls2], [gs0, gs1, gs2], [ws0, ws1, ws2]
        cid = lax.axis_index("c")
        sid = lax.axis_index("s")
        row0 = sid * rows_per_tile

        def fire_idx(t, s):
            pltpu.async_copy(sd2_hbm.at[cid, sid + t * _NS], idx_v[s], lsem[s])

        def wait_idx(s):
            pltpu.make_async_copy(sd2_hbm.at[0, 0], idx_v[s], lsem[s]).wait()

        def fire_gather(s):
            pltpu.async_copy(table.at[idx_v[s]], buf[s], gsem[s])

        def wait_gather(s):
            pltpu.make_async_copy(table.at[idx_v[s]], buf[s], gsem[s]).wait()

        def fire_wo(t, s):
            base = (sid + t * _NS) * _CHUNK
            pltpu.async_copy(buf[s], g2_hbm.at[cid, pl.ds(base, _CHUNK)], wsem[s])

        def wait_wo(s):
            pltpu.make_async_copy(buf[s], g2_hbm.at[0, pl.ds(0, _CHUNK)], wsem[s]).wait()

        pltpu.sync_copy(pt_hbm.at[cid, pl.ds(row0, rows_per_tile)],
                        table.at[pl.ds(row0, rows_per_tile)])
        plsc.subcore_barrier()

        fire_idx(0, 0)
        wait_idx(0); fire_gather(0); fire_idx(1, 1)
        wait_idx(1); fire_gather(1); wait_gather(0); fire_wo(0, 0); fire_idx(2, 2)
        wait_idx(2); fire_gather(2); wait_gather(1); fire_wo(1, 1); fire_idx(3, 0)

        def loop_body(k, _):
            for s in range(3):
                t = 3 * k + s
                s1 = (s + 1) % 3
                s2 = (s + 2) % 3
                wait_idx(s)
                wait_wo(s)
                fire_gather(s)
                wait_gather(s2)
                fire_wo(t - 1, s2)

                @pl.when(t + 1 < per_tile)
                def _():
                    fire_idx(t + 1, s1)
            return 0

        lax.fori_loop(1, n_k, loop_body, 0)

        wait_gather(2)
        fire_wo(per_tile - 1, 2)
        wait_wo(0); wait_wo(1); wait_wo(2)

        @pl.when(sid < n_chunks - n_even)
        def _():
            c = n_even + sid
            pltpu.sync_copy(sd2_hbm.at[cid, c], idx_v[0])
            fire_gather(0)
            wait_gather(0)
            pltpu.sync_copy(buf[0], g2_hbm.at[cid, pl.ds(c * _CHUNK, _CHUNK)])

    return gather_k


def _make_scatter(E, N_pad):
    n_chunks = E // _CHUNK
    n_even = (n_chunks // _NW) * _NW
    per_tile = n_even // _NW
    n_k = per_tile // 3
    rows_per_tile = N_pad // _NS

    @functools.partial(
        pl.kernel,
        out_type=jax.ShapeDtypeStruct((_NC, N_pad, 128), jnp.float32),
        mesh=_sc_mesh(),
        scratch_types=(
            [pltpu.VMEM((_CHUNK,), jnp.int32)] * 3
            + [pltpu.VMEM((_CHUNK, 128), jnp.float32)] * 3
            + [pltpu.VMEM_SHARED((N_pad, 128), jnp.float32)]
            + [pltpu.SemaphoreType.DMA] * 6
        ),
    )
    def scatter_k(he_hbm, dst_hbm, zeros_hbm, parts_hbm,
                  ix0, ix1, ix2, rw0, rw1, rw2, acc,
                  ls0, ls1, ls2, ss0, ss1, ss2):
        idx_v, rows_v = [ix0, ix1, ix2], [rw0, rw1, rw2]
        lsem, ssem = [ls0, ls1, ls2], [ss0, ss1, ss2]
        cid = lax.axis_index("c")
        sid = lax.axis_index("s")
        wid = cid * _NS + sid
        row0 = sid * rows_per_tile

        def fire_load(t, s):
            base = (wid + t * _NW) * _CHUNK
            pltpu.async_copy(dst_hbm.at[pl.ds(base, _CHUNK)], idx_v[s], lsem[s])
            pltpu.async_copy(he_hbm.at[pl.ds(base, _CHUNK)], rows_v[s], lsem[s])

        def wait_load(s):
            pltpu.make_async_copy(dst_hbm.at[pl.ds(0, _CHUNK)], idx_v[s], lsem[s]).wait()
            pltpu.make_async_copy(he_hbm.at[pl.ds(0, _CHUNK)], rows_v[s], lsem[s]).wait()

        def fire_scatter(s):
            pltpu.async_copy(rows_v[s], acc.at[idx_v[s]], ssem[s], add=True)

        def wait_scatter(s):
            pltpu.make_async_copy(rows_v[s], acc.at[idx_v[s]], ssem[s]).wait()

        fire_load(0, 0)
        pltpu.sync_copy(zeros_hbm.at[pl.ds(row0, rows_per_tile)],
                        acc.at[pl.ds(row0, rows_per_tile)])
        plsc.subcore_barrier()

        wait_load(0); fire_load(1, 1); fire_scatter(0)
        wait_load(1); fire_load(2, 2); fire_scatter(1)
        wait_load(2); wait_scatter(0); fire_load(3, 0); fire_scatter(2)

        def loop_body(k, _):
            for s in range(3):
                t = 3 * k + s
                s2 = (s + 1) % 3
                wait_load(s)
                wait_scatter(s2)

                @pl.when(t + 1 < per_tile)
                def _():
                    fire_load(t + 1, s2)

                fire_scatter(s)
            return 0

        lax.fori_loop(1, n_k, loop_body, 0)
        wait_scatter(1)
        wait_scatter(2)

        @pl.when(wid < n_chunks - n_even)
        def _():
            base = (n_even + wid) * _CHUNK
            pltpu.sync_copy(dst_hbm.at[pl.ds(base, _CHUNK)], idx_v[0])
            pltpu.sync_copy(he_hbm.at[pl.ds(base, _CHUNK)], rows_v[0])
            pltpu.sync_copy(rows_v[0], acc.at[idx_v[0]], add=True)

        plsc.subcore_barrier()
        pltpu.sync_copy(acc.at[pl.ds(row0, rows_per_tile)],
                        parts_hbm.at[cid, pl.ds(row0, rows_per_tile)])

    return scatter_k


def _b(v):
    return v.reshape(1, -1)


def kernel(node_feats, edge_index, edge_feats, params):
    N, node_in = node_feats.shape
    E, edge_in = edge_feats.shape
    src = edge_index[0]
    dst = edge_index[1]

    rb_n = 1000
    rb_e = 1000

    def enc(x, p, rb, rows, din):
        return _call_rows(
            _enc_body, rows, rb,
            [x, p["W1"], _b(p["b1"]), p["W2"], _b(p["b2"]),
             _b(p["gamma"]), _b(p["beta"])],
            [din], [128])

    h_n = enc(node_feats, params["node_enc"], rb_n, N, node_in)
    E2 = E // 2
    dst_h = [dst[:E2], dst[E2:]]
    sd2_h = [jnp.stack([src[:E2].reshape(-1, _CHUNK), dst[:E2].reshape(-1, _CHUNK)]),
             jnp.stack([src[E2:].reshape(-1, _CHUNK), dst[E2:].reshape(-1, _CHUNK)])]
    h_e = [enc(edge_feats[:E2], params["edge_enc"], rb_e, E2, edge_in),
           enc(edge_feats[E2:], params["edge_enc"], rb_e, E2, edge_in)]

    n_pad = ((N + 8 * _NS - 1) // (8 * _NS)) * (8 * _NS)
    rb_p = n_pad // 16
    gather_k = _make_gather(E2, n_pad)
    scatter_k = _make_scatter(E2, n_pad)
    zeros_n = jnp.zeros((n_pad, 128), jnp.float32)

    for i in range(6):
        pe = params["edge_mlps"][i]
        pn = params["node_mlps"][i]
        w1s = pe["W1"][0:128]
        w1d = pe["W1"][128:256]
        w1e = pe["W1"][256:384]

        ps, pd = _call_rows(_proj_body, n_pad, rb_p, [h_n, w1s, w1d],
                            [128], [128, 128])
        pt = jnp.stack([ps, pd])
        parts = [None, None]
        for h in range(2):
            g2 = gather_k(pt, sd2_h[h])
            h_e[h] = _call_rows(
                _edge_body, E2, rb_e,
                [h_e[h], g2[0], g2[1], w1e, _b(pe["b1"]), pe["W2"], _b(pe["b2"]),
                 _b(pe["gamma"]), _b(pe["beta"])],
                [128, 128, 128], [128])
            parts[h] = scatter_k(h_e[h], dst_h[h], zeros_n)
        h_n = _call_rows(
            _node_body, N, rb_n,
            [h_n, parts[0][0, :N], parts[0][1, :N], parts[1][0, :N],
             parts[1][1, :N], pn["W1"][0:128], pn["W1"][128:256],
             _b(pn["b1"]), pn["W2"], _b(pn["b2"]),
             _b(pn["gamma"]), _b(pn["beta"])],
            [128, 128, 128, 128, 128], [128])

    dec = params["dec"]
    w2p = jnp.zeros((128, 128), jnp.float32).at[:, :dec["W2"].shape[1]].set(dec["W2"])
    b2p = jnp.zeros((128,), jnp.float32).at[:dec["b2"].shape[0]].set(dec["b2"])
    out = _call_rows(_dec_body, N, rb_n,
                     [h_n, dec["W1"], _b(dec["b1"]), w2p, _b(b2p)],
                     [128], [128])
    return out[:, :dec["W2"].shape[1]]


# speedup vs baseline: 1.3533x; 1.3533x over previous
import functools

import jax
import jax.numpy as jnp
from jax import lax
from jax.experimental import pallas as pl
from jax.experimental.pallas import tpu as pltpu
from jax.experimental.pallas import tpu_sc as plsc

_NC = 2
_NS = 16
_NW = _NC * _NS
_LN_EPS = 1e-5
_CHUNK = 128


def _ln(y, gamma, beta):
    mu = jnp.mean(y, axis=-1, keepdims=True)
    var = jnp.mean((y - mu) ** 2, axis=-1, keepdims=True)
    return (y - mu) * lax.rsqrt(var + _LN_EPS) * gamma + beta


def _silu(x):
    return x * jax.nn.sigmoid(x)


def _enc_body(x_ref, w1_ref, b1_ref, w2_ref, b2_ref, g_ref, be_ref, o_ref):
    h = jnp.dot(x_ref[...], w1_ref[...], preferred_element_type=jnp.float32)
    h = _silu(h + b1_ref[...])
    y = jnp.dot(h, w2_ref[...], preferred_element_type=jnp.float32) + b2_ref[...]
    o_ref[...] = _ln(y, g_ref[...], be_ref[...])


def _edge_body(he_ref, gadd_ref, w1_ref, b1_ref, w2_ref, b2_ref, g_ref,
               be_ref, o_ref):
    he = he_ref[...]
    h = jnp.dot(he, w1_ref[...], preferred_element_type=jnp.float32)
    h = _silu(h + gadd_ref[...] + b1_ref[...])
    y = jnp.dot(h, w2_ref[...], preferred_element_type=jnp.float32) + b2_ref[...]
    o_ref[...] = he + _ln(y, g_ref[...], be_ref[...])


def _node_body(hn_ref, p0_ref, p1_ref, p2_ref, p3_ref, w1a_ref, w1b_ref,
               b1_ref, w2_ref, b2_ref, g_ref, be_ref, o_ref):
    hn = hn_ref[...]
    agg = (p0_ref[...] + p1_ref[...]) + (p2_ref[...] + p3_ref[...])
    h = jnp.dot(hn, w1a_ref[...], preferred_element_type=jnp.float32)
    h = h + jnp.dot(agg, w1b_ref[...], preferred_element_type=jnp.float32)
    h = _silu(h + b1_ref[...])
    y = jnp.dot(h, w2_ref[...], preferred_element_type=jnp.float32) + b2_ref[...]
    o_ref[...] = hn + _ln(y, g_ref[...], be_ref[...])


def _proj_body(hn_ref, ws_ref, wd_ref, ps_ref, pd_ref):
    hn = hn_ref[...]
    ps_ref[...] = jnp.dot(hn, ws_ref[...], preferred_element_type=jnp.float32)
    pd_ref[...] = jnp.dot(hn, wd_ref[...], preferred_element_type=jnp.float32)


def _dec_body(hn_ref, w1_ref, b1_ref, w2_ref, b2_ref, o_ref):
    h = jnp.dot(hn_ref[...], w1_ref[...], preferred_element_type=jnp.float32)
    h = _silu(h + b1_ref[...])
    o_ref[...] = jnp.dot(h, w2_ref[...], preferred_element_type=jnp.float32) + b2_ref[...]


def _row_spec(rb, d):
    return pl.BlockSpec((rb, d), lambda i: (i, 0))


def _full_spec(shape):
    return pl.BlockSpec(shape, lambda i: tuple(0 for _ in shape))


def _call_rows(body, rows, rb, ins, in_dims, out_dims):
    n_row = len(in_dims)
    specs = [_row_spec(rb, d) for d in in_dims]
    specs += [_full_spec(x.shape) for x in ins[n_row:]]
    multi = len(out_dims) > 1
    out_shape = [jax.ShapeDtypeStruct((rows, d), jnp.float32) for d in out_dims]
    out_spec = [_row_spec(rb, d) for d in out_dims]
    if not multi:
        out_shape, out_spec = out_shape[0], out_spec[0]
    return pl.pallas_call(
        body,
        grid=(rows // rb,),
        in_specs=specs,
        out_specs=out_spec,
        out_shape=out_shape,
    )(*ins)


def _sc_mesh():
    return plsc.VectorSubcoreMesh(core_axis_name="c", subcore_axis_name="s",
                                  num_cores=_NC, num_subcores=_NS)


def _make_gather(E, N_pad):
    n_chunks = E // _CHUNK
    n_even = (n_chunks // _NW) * _NW
    per_tile = n_even // _NW
    n_k = per_tile // 3

    @functools.partial(
        pl.kernel,
        out_type=jax.ShapeDtypeStruct((E, 128), jnp.float32),
        mesh=_sc_mesh(),
        scratch_types=(
            [pltpu.VMEM((2, _CHUNK), jnp.int32)] * 3
            + [pltpu.VMEM((_CHUNK, 128), jnp.float32)] * 6
            + [pltpu.SemaphoreType.DMA] * 9
        ),
    )
    def gather_k(p2_hbm, sd2_hbm, g_hbm,
                 ix0, ix1, ix2, ba0, ba1, ba2, bb0, bb1, bb2,
                 ls0, ls1, ls2, gs0, gs1, gs2, ws0, ws1, ws2):
        idx_v = [ix0, ix1, ix2]
        buf_a, buf_b = [ba0, ba1, ba2], [bb0, bb1, bb2]
        lsem, gsem, wsem = [ls0, ls1, ls2], [gs0, gs1, gs2], [ws0, ws1, ws2]
        wid = lax.axis_index("c") * _NS + lax.axis_index("s")

        def fire_idx(t, s):
            pltpu.async_copy(sd2_hbm.at[wid + t * _NW], idx_v[s], lsem[s])

        def wait_idx(s):
            pltpu.make_async_copy(sd2_hbm.at[0], idx_v[s], lsem[s]).wait()

        def fire_gather(s):
            pltpu.async_copy(p2_hbm.at[idx_v[s].at[0]], buf_a[s], gsem[s])
            pltpu.async_copy(p2_hbm.at[idx_v[s].at[1]], buf_b[s], gsem[s])

        def wait_gather(s):
            pltpu.make_async_copy(p2_hbm.at[idx_v[s].at[0]], buf_a[s], gsem[s]).wait()
            pltpu.make_async_copy(p2_hbm.at[idx_v[s].at[1]], buf_b[s], gsem[s]).wait()

        def add_slot(s):
            def body(i, _):
                for kk in range(8):
                    sl = pl.ds(kk * 16, 16)
                    buf_a[s][i, sl] = buf_a[s][i, sl] + buf_b[s][i, sl]
                return 0
            lax.fori_loop(0, _CHUNK, body, 0)

        def fire_wo(t, s):
            base = (wid + t * _NW) * _CHUNK
            pltpu.async_copy(buf_a[s], g_hbm.at[pl.ds(base, _CHUNK)], wsem[s])

        def wait_wo(s):
            pltpu.make_async_copy(buf_a[s], g_hbm.at[pl.ds(0, _CHUNK)], wsem[s]).wait()

        fire_idx(0, 0)
        wait_idx(0); fire_gather(0); fire_idx(1, 1)
        wait_idx(1); fire_gather(1); fire_idx(2, 2)
        wait_idx(2); fire_gather(2)
        wait_gather(0); add_slot(0); fire_wo(0, 0); fire_idx(3, 0)

        def loop_body(k, _):
            for s in range(3):
                t = 3 * k + s
                s2 = (s + 1) % 3
                wait_idx(s)
                wait_wo(s)
                fire_gather(s)
                wait_gather(s2)
                add_slot(s2)
                fire_wo(t - 2, s2)

                @pl.when(t + 1 < per_tile)
                def _():
                    fire_idx(t + 1, s2)
            return 0

        lax.fori_loop(1, n_k, loop_body, 0)

        wait_gather(1); add_slot(1); fire_wo(per_tile - 2, 1)
        wait_gather(2); add_slot(2); fire_wo(per_tile - 1, 2)
        wait_wo(0); wait_wo(1); wait_wo(2)

        @pl.when(wid < n_chunks - n_even)
        def _():
            c = n_even + wid
            pltpu.sync_copy(sd2_hbm.at[c], idx_v[0])
            fire_gather(0)
            wait_gather(0)
            add_slot(0)
            pltpu.sync_copy(buf_a[0], g_hbm.at[pl.ds(c * _CHUNK, _CHUNK)])

    return gather_k


def _make_scatter(E, N_pad):
    n_chunks = E // _CHUNK
    n_even = (n_chunks // _NW) * _NW
    per_tile = n_even // _NW
    n_k = per_tile // 3
    rows_per_tile = N_pad // _NS

    @functools.partial(
        pl.kernel,
        out_type=jax.ShapeDtypeStruct((_NC, N_pad, 128), jnp.float32),
        mesh=_sc_mesh(),
        scratch_types=(
            [pltpu.VMEM((_CHUNK,), jnp.int32)] * 3
            + [pltpu.VMEM((_CHUNK, 128), jnp.float32)] * 3
            + [pltpu.VMEM_SHARED((N_pad, 128), jnp.float32)]
            + [pltpu.SemaphoreType.DMA] * 6
        ),
    )
    def scatter_k(he_hbm, dst_hbm, zeros_hbm, parts_hbm,
                  ix0, ix1, ix2, rw0, rw1, rw2, acc,
                  ls0, ls1, ls2, ss0, ss1, ss2):
        idx_v, rows_v = [ix0, ix1, ix2], [rw0, rw1, rw2]
        lsem, ssem = [ls0, ls1, ls2], [ss0, ss1, ss2]
        cid = lax.axis_index("c")
        sid = lax.axis_index("s")
        wid = cid * _NS + sid
        row0 = sid * rows_per_tile

        def fire_load(t, s):
            base = (wid + t * _NW) * _CHUNK
            pltpu.async_copy(dst_hbm.at[pl.ds(base, _CHUNK)], idx_v[s], lsem[s])
            pltpu.async_copy(he_hbm.at[pl.ds(base, _CHUNK)], rows_v[s], lsem[s])

        def wait_load(s):
            pltpu.make_async_copy(dst_hbm.at[pl.ds(0, _CHUNK)], idx_v[s], lsem[s]).wait()
            pltpu.make_async_copy(he_hbm.at[pl.ds(0, _CHUNK)], rows_v[s], lsem[s]).wait()

        def fire_scatter(s):
            pltpu.async_copy(rows_v[s], acc.at[idx_v[s]], ssem[s], add=True)

        def wait_scatter(s):
            pltpu.make_async_copy(rows_v[s], acc.at[idx_v[s]], ssem[s]).wait()

        fire_load(0, 0)
        pltpu.sync_copy(zeros_hbm.at[pl.ds(row0, rows_per_tile)],
                        acc.at[pl.ds(row0, rows_per_tile)])
        plsc.subcore_barrier()

        wait_load(0); fire_load(1, 1); fire_scatter(0)
        wait_load(1); fire_load(2, 2); fire_scatter(1)
        wait_load(2); wait_scatter(0); fire_load(3, 0); fire_scatter(2)

        def loop_body(k, _):
            for s in range(3):
                t = 3 * k + s
                s2 = (s + 1) % 3
                wait_load(s)
                wait_scatter(s2)

                @pl.when(t + 1 < per_tile)
                def _():
                    fire_load(t + 1, s2)

                fire_scatter(s)
            return 0

        lax.fori_loop(1, n_k, loop_body, 0)
        wait_scatter(1)
        wait_scatter(2)

        @pl.when(wid < n_chunks - n_even)
        def _():
            base = (n_even + wid) * _CHUNK
            pltpu.sync_copy(dst_hbm.at[pl.ds(base, _CHUNK)], idx_v[0])
            pltpu.sync_copy(he_hbm.at[pl.ds(base, _CHUNK)], rows_v[0])
            pltpu.sync_copy(rows_v[0], acc.at[idx_v[0]], add=True)

        plsc.subcore_barrier()
        pltpu.sync_copy(acc.at[pl.ds(row0, rows_per_tile)],
                        parts_hbm.at[cid, pl.ds(row0, rows_per_tile)])

    return scatter_k


def _b(v):
    return v.reshape(1, -1)


def kernel(node_feats, edge_index, edge_feats, params):
    N, node_in = node_feats.shape
    E, edge_in = edge_feats.shape
    src = edge_index[0]
    dst = edge_index[1]

    rb_n = 1000
    rb_e = 1000

    def enc(x, p, rb, rows, din):
        return _call_rows(
            _enc_body, rows, rb,
            [x, p["W1"], _b(p["b1"]), p["W2"], _b(p["b2"]),
             _b(p["gamma"]), _b(p["beta"])],
            [din], [128])

    h_n = enc(node_feats, params["node_enc"], rb_n, N, node_in)
    E2 = E // 2
    n_pad = ((N + 8 * _NS - 1) // (8 * _NS)) * (8 * _NS)
    rb_p = n_pad // 16
    dst_h = [dst[:E2], dst[E2:]]

    def mk_sd2(s_half, d_half):
        return jnp.concatenate(
            [s_half.reshape(-1, 1, _CHUNK),
             d_half.reshape(-1, 1, _CHUNK) + n_pad], axis=1)

    sd2_h = [mk_sd2(src[:E2], dst[:E2]), mk_sd2(src[E2:], dst[E2:])]
    h_e = [enc(edge_feats[:E2], params["edge_enc"], rb_e, E2, edge_in),
           enc(edge_feats[E2:], params["edge_enc"], rb_e, E2, edge_in)]

    gather_k = _make_gather(E2, n_pad)
    scatter_k = _make_scatter(E2, n_pad)
    zeros_n = jnp.zeros((n_pad, 128), jnp.float32)

    for i in range(6):
        pe = params["edge_mlps"][i]
        pn = params["node_mlps"][i]
        w1s = pe["W1"][0:128]
        w1d = pe["W1"][128:256]
        w1e = pe["W1"][256:384]

        ps, pd = _call_rows(_proj_body, n_pad, rb_p, [h_n, w1s, w1d],
                            [128], [128, 128])
        p2 = jnp.concatenate([ps, pd], axis=0)
        parts = [None, None]
        for h in range(2):
            g = gather_k(p2, sd2_h[h])
            h_e[h] = _call_rows(
                _edge_body, E2, rb_e,
                [h_e[h], g, w1e, _b(pe["b1"]), pe["W2"], _b(pe["b2"]),
                 _b(pe["gamma"]), _b(pe["beta"])],
                [128, 128], [128])
            parts[h] = scatter_k(h_e[h], dst_h[h], zeros_n)
        h_n = _call_rows(
            _node_body, N, rb_n,
            [h_n, parts[0][0, :N], parts[0][1, :N], parts[1][0, :N],
             parts[1][1, :N], pn["W1"][0:128], pn["W1"][128:256],
             _b(pn["b1"]), pn["W2"], _b(pn["b2"]),
             _b(pn["gamma"]), _b(pn["beta"])],
            [128, 128, 128, 128, 128], [128])

    dec = params["dec"]
    w2p = jnp.zeros((128, 128), jnp.float32).at[:, :dec["W2"].shape[1]].set(dec["W2"])
    b2p = jnp.zeros((128,), jnp.float32).at[:dec["b2"].shape[0]].set(dec["b2"])
    out = _call_rows(_dec_body, N, rb_n,
                     [h_n, dec["W1"], _b(dec["b1"]), w2p, _b(b2p)],
                     [128], [128])
    return out[:, :dec["W2"].shape[1]]


# speedup vs baseline: 1.6876x; 1.2471x over previous
import functools

import jax
import jax.numpy as jnp
from jax import lax
from jax.experimental import pallas as pl
from jax.experimental.pallas import tpu as pltpu
from jax.experimental.pallas import tpu_sc as plsc

_NC = 2
_NS = 16
_NW = _NC * _NS
_LN_EPS = 1e-5
_CHUNK = 128


def _ln_centered(d, gamma, beta):
    var = jnp.mean(d * d, axis=-1, keepdims=True)
    return d * (lax.rsqrt(var + _LN_EPS) * gamma) + beta


def _silu(x):
    return x * jax.nn.sigmoid(x)


def _enc_body(x_ref, w1_ref, b1_ref, w2_ref, b2_ref, g_ref, be_ref, o_ref):
    h = jnp.dot(x_ref[...], w1_ref[...], preferred_element_type=jnp.float32)
    h = _silu(h + b1_ref[...])
    d = jnp.dot(h, w2_ref[...], preferred_element_type=jnp.float32) + b2_ref[...]
    o_ref[...] = _ln_centered(d, g_ref[...], be_ref[...])


def _edge_body(he_ref, gadd_ref, w1_ref, b1_ref, w2_ref, b2_ref, g_ref,
               be_ref, o_ref):
    he = he_ref[...]
    h = jnp.dot(he, w1_ref[...], preferred_element_type=jnp.float32)
    h = _silu(h + gadd_ref[...] + b1_ref[...])
    d = jnp.dot(h, w2_ref[...], preferred_element_type=jnp.float32) + b2_ref[...]
    o_ref[...] = he + _ln_centered(d, g_ref[...], be_ref[...])


def _node_body(hn_ref, p0_ref, p1_ref, p2_ref, p3_ref, w1a_ref, w1b_ref,
               b1_ref, w2_ref, b2_ref, g_ref, be_ref, o_ref):
    hn = hn_ref[...]
    agg = (p0_ref[...] + p1_ref[...]) + (p2_ref[...] + p3_ref[...])
    h = jnp.dot(hn, w1a_ref[...], preferred_element_type=jnp.float32)
    h = h + jnp.dot(agg, w1b_ref[...], preferred_element_type=jnp.float32)
    h = _silu(h + b1_ref[...])
    d = jnp.dot(h, w2_ref[...], preferred_element_type=jnp.float32) + b2_ref[...]
    o_ref[...] = hn + _ln_centered(d, g_ref[...], be_ref[...])


def _proj_body(hn_ref, ws_ref, wd_ref, ps_ref, pd_ref):
    hn = hn_ref[...]
    ps_ref[...] = jnp.dot(hn, ws_ref[...], preferred_element_type=jnp.float32)
    pd_ref[...] = jnp.dot(hn, wd_ref[...], preferred_element_type=jnp.float32)


def _dec_body(hn_ref, w1_ref, b1_ref, w2_ref, b2_ref, o_ref):
    h = jnp.dot(hn_ref[...], w1_ref[...], preferred_element_type=jnp.float32)
    h = _silu(h + b1_ref[...])
    o_ref[...] = jnp.dot(h, w2_ref[...], preferred_element_type=jnp.float32) + b2_ref[...]


def _row_spec(rb, d):
    return pl.BlockSpec((rb, d), lambda i: (i, 0))


def _full_spec(shape):
    return pl.BlockSpec(shape, lambda i: tuple(0 for _ in shape))


def _call_rows(body, rows, rb, ins, in_dims, out_dims):
    n_row = len(in_dims)
    specs = [_row_spec(rb, d) for d in in_dims]
    specs += [_full_spec(x.shape) for x in ins[n_row:]]
    multi = len(out_dims) > 1
    out_shape = [jax.ShapeDtypeStruct((rows, d), jnp.float32) for d in out_dims]
    out_spec = [_row_spec(rb, d) for d in out_dims]
    if not multi:
        out_shape, out_spec = out_shape[0], out_spec[0]
    return pl.pallas_call(
        body,
        grid=(rows // rb,),
        in_specs=specs,
        out_specs=out_spec,
        out_shape=out_shape,
    )(*ins)


def _sc_mesh():
    return plsc.VectorSubcoreMesh(core_axis_name="c", subcore_axis_name="s",
                                  num_cores=_NC, num_subcores=_NS)


def _make_gather(E, N_pad):
    n_chunks = E // _CHUNK
    n_even = (n_chunks // _NW) * _NW
    per_tile = n_even // _NW
    n_k = per_tile // 3

    @functools.partial(
        pl.kernel,
        out_type=jax.ShapeDtypeStruct((E, 128), jnp.float32),
        mesh=_sc_mesh(),
        scratch_types=(
            [pltpu.VMEM((2, _CHUNK), jnp.int32)] * 3
            + [pltpu.VMEM((_CHUNK, 128), jnp.float32)] * 6
            + [pltpu.SemaphoreType.DMA] * 9
        ),
    )
    def gather_k(ps_hbm, pd_hbm, sd2_hbm, g_hbm,
                 ix0, ix1, ix2, ba0, ba1, ba2, bb0, bb1, bb2,
                 ls0, ls1, ls2, gs0, gs1, gs2, ws0, ws1, ws2):
        idx_v = [ix0, ix1, ix2]
        buf_a, buf_b = [ba0, ba1, ba2], [bb0, bb1, bb2]
        lsem, gsem, wsem = [ls0, ls1, ls2], [gs0, gs1, gs2], [ws0, ws1, ws2]
        wid = lax.axis_index("c") * _NS + lax.axis_index("s")

        def fire_idx(t, s):
            pltpu.async_copy(sd2_hbm.at[wid + t * _NW], idx_v[s], lsem[s])

        def wait_idx(s):
            pltpu.make_async_copy(sd2_hbm.at[0], idx_v[s], lsem[s]).wait()

        def fire_gather(s):
            pltpu.async_copy(ps_hbm.at[idx_v[s].at[0]], buf_a[s], gsem[s])
            pltpu.async_copy(pd_hbm.at[idx_v[s].at[1]], buf_b[s], gsem[s])

        def wait_gather(s):
            pltpu.make_async_copy(ps_hbm.at[idx_v[s].at[0]], buf_a[s], gsem[s]).wait()
            pltpu.make_async_copy(pd_hbm.at[idx_v[s].at[1]], buf_b[s], gsem[s]).wait()

        def add_slot(s):
            def body(i, _):
                for kk in range(8):
                    sl = pl.ds(kk * 16, 16)
                    buf_a[s][i, sl] = buf_a[s][i, sl] + buf_b[s][i, sl]
                return 0
            lax.fori_loop(0, _CHUNK, body, 0)

        def fire_wo(t, s):
            base = (wid + t * _NW) * _CHUNK
            pltpu.async_copy(buf_a[s], g_hbm.at[pl.ds(base, _CHUNK)], wsem[s])

        def wait_wo(s):
            pltpu.make_async_copy(buf_a[s], g_hbm.at[pl.ds(0, _CHUNK)], wsem[s]).wait()

        fire_idx(0, 0)
        wait_idx(0); fire_gather(0); fire_idx(1, 1)
        wait_idx(1); fire_gather(1); fire_idx(2, 2)
        wait_idx(2); fire_gather(2)
        wait_gather(0); add_slot(0); fire_wo(0, 0); fire_idx(3, 0)

        def loop_body(k, _):
            for s in range(3):
                t = 3 * k + s
                s2 = (s + 1) % 3
                wait_idx(s)
                wait_wo(s)
                fire_gather(s)
                wait_gather(s2)
                add_slot(s2)
                fire_wo(t - 2, s2)

                @pl.when(t + 1 < per_tile)
                def _():
                    fire_idx(t + 1, s2)
            return 0

        lax.fori_loop(1, n_k, loop_body, 0)

        wait_gather(1); add_slot(1); fire_wo(per_tile - 2, 1)
        wait_gather(2); add_slot(2); fire_wo(per_tile - 1, 2)
        wait_wo(0); wait_wo(1); wait_wo(2)

        @pl.when(wid < n_chunks - n_even)
        def _():
            c = n_even + wid
            pltpu.sync_copy(sd2_hbm.at[c], idx_v[0])
            fire_gather(0)
            wait_gather(0)
            add_slot(0)
            pltpu.sync_copy(buf_a[0], g_hbm.at[pl.ds(c * _CHUNK, _CHUNK)])

    return gather_k


def _make_scatter(E, N_pad):
    n_chunks = E // _CHUNK
    n_even = (n_chunks // _NW) * _NW
    per_tile = n_even // _NW
    n_k = per_tile // 3
    rows_per_tile = N_pad // _NS

    @functools.partial(
        pl.kernel,
        out_type=jax.ShapeDtypeStruct((_NC, N_pad, 128), jnp.float32),
        mesh=_sc_mesh(),
        scratch_types=(
            [pltpu.VMEM((_CHUNK,), jnp.int32)] * 3
            + [pltpu.VMEM((_CHUNK, 128), jnp.float32)] * 3
            + [pltpu.VMEM_SHARED((N_pad, 128), jnp.float32)]
            + [pltpu.SemaphoreType.DMA] * 6
        ),
    )
    def scatter_k(he_hbm, dst_hbm, zeros_hbm, parts_hbm,
                  ix0, ix1, ix2, rw0, rw1, rw2, acc,
                  ls0, ls1, ls2, ss0, ss1, ss2):
        idx_v, rows_v = [ix0, ix1, ix2], [rw0, rw1, rw2]
        lsem, ssem = [ls0, ls1, ls2], [ss0, ss1, ss2]
        cid = lax.axis_index("c")
        sid = lax.axis_index("s")
        wid = cid * _NS + sid
        row0 = sid * rows_per_tile

        def fire_load(t, s):
            base = (wid + t * _NW) * _CHUNK
            pltpu.async_copy(dst_hbm.at[pl.ds(base, _CHUNK)], idx_v[s], lsem[s])
            pltpu.async_copy(he_hbm.at[pl.ds(base, _CHUNK)], rows_v[s], lsem[s])

        def wait_load(s):
            pltpu.make_async_copy(dst_hbm.at[pl.ds(0, _CHUNK)], idx_v[s], lsem[s]).wait()
            pltpu.make_async_copy(he_hbm.at[pl.ds(0, _CHUNK)], rows_v[s], lsem[s]).wait()

        def fire_scatter(s):
            pltpu.async_copy(rows_v[s], acc.at[idx_v[s]], ssem[s], add=True)

        def wait_scatter(s):
            pltpu.make_async_copy(rows_v[s], acc.at[idx_v[s]], ssem[s]).wait()

        fire_load(0, 0)
        pltpu.sync_copy(zeros_hbm.at[pl.ds(row0, rows_per_tile)],
                        acc.at[pl.ds(row0, rows_per_tile)])
        plsc.subcore_barrier()

        wait_load(0); fire_load(1, 1); fire_scatter(0)
        wait_load(1); fire_load(2, 2); fire_scatter(1)
        wait_load(2); wait_scatter(0); fire_load(3, 0); fire_scatter(2)

        def loop_body(k, _):
            for s in range(3):
                t = 3 * k + s
                s2 = (s + 1) % 3
                wait_load(s)
                wait_scatter(s2)

                @pl.when(t + 1 < per_tile)
                def _():
                    fire_load(t + 1, s2)

                fire_scatter(s)
            return 0

        lax.fori_loop(1, n_k, loop_body, 0)
        wait_scatter(1)
        wait_scatter(2)

        @pl.when(wid < n_chunks - n_even)
        def _():
            base = (n_even + wid) * _CHUNK
            pltpu.sync_copy(dst_hbm.at[pl.ds(base, _CHUNK)], idx_v[0])
            pltpu.sync_copy(he_hbm.at[pl.ds(base, _CHUNK)], rows_v[0])
            pltpu.sync_copy(rows_v[0], acc.at[idx_v[0]], add=True)

        plsc.subcore_barrier()
        pltpu.sync_copy(acc.at[pl.ds(row0, rows_per_tile)],
                        parts_hbm.at[cid, pl.ds(row0, rows_per_tile)])

    return scatter_k


def _b(v):
    return v.reshape(1, -1)


def kernel(node_feats, edge_index, edge_feats, params):
    N, node_in = node_feats.shape
    E, edge_in = edge_feats.shape
    src = edge_index[0]
    dst = edge_index[1]

    rb_n = 1000
    rb_e = 2000

    def center(p):
        w2 = p["W2"] - jnp.mean(p["W2"], axis=1, keepdims=True)
        b2 = p["b2"] - jnp.mean(p["b2"])
        return w2, b2

    def enc(x, p, rb, rows, din):
        w2c, b2c = center(p)
        return _call_rows(
            _enc_body, rows, rb,
            [x, p["W1"], _b(p["b1"]), w2c, _b(b2c),
             _b(p["gamma"]), _b(p["beta"])],
            [din], [128])

    h_n = enc(node_feats, params["node_enc"], rb_n, N, node_in)
    E2 = E // 2
    n_pad = ((N + 8 * _NS - 1) // (8 * _NS)) * (8 * _NS)
    rb_p = n_pad // 16
    dst_h = [dst[:E2], dst[E2:]]

    def mk_sd2(s_half, d_half):
        return jnp.concatenate(
            [s_half.reshape(-1, 1, _CHUNK),
             d_half.reshape(-1, 1, _CHUNK)], axis=1)

    sd2_h = [mk_sd2(src[:E2], dst[:E2]), mk_sd2(src[E2:], dst[E2:])]
    h_e = [enc(edge_feats[:E2], params["edge_enc"], rb_e, E2, edge_in),
           enc(edge_feats[E2:], params["edge_enc"], rb_e, E2, edge_in)]

    gather_k = _make_gather(E2, n_pad)
    scatter_k = _make_scatter(E2, n_pad)
    zeros_n = jnp.zeros((n_pad, 128), jnp.float32)

    for i in range(6):
        pe = params["edge_mlps"][i]
        pn = params["node_mlps"][i]
        w1s = pe["W1"][0:128]
        w1d = pe["W1"][128:256]
        w1e = pe["W1"][256:384]

        we2c, we2bc = center(pe)
        wn2c, wn2bc = center(pn)

        ps, pd = _call_rows(_proj_body, n_pad, rb_p, [h_n, w1s, w1d],
                            [128], [128, 128])
        parts = [None, None]
        for h in range(2):
            g = gather_k(ps, pd, sd2_h[h])
            h_e[h] = _call_rows(
                _edge_body, E2, rb_e,
                [h_e[h], g, w1e, _b(pe["b1"]), we2c, _b(we2bc),
                 _b(pe["gamma"]), _b(pe["beta"])],
                [128, 128], [128])
            parts[h] = scatter_k(h_e[h], dst_h[h], zeros_n)
        h_n = _call_rows(
            _node_body, N, rb_n,
            [h_n, parts[0][0, :N], parts[0][1, :N], parts[1][0, :N],
             parts[1][1, :N], pn["W1"][0:128], pn["W1"][128:256],
             _b(pn["b1"]), wn2c, _b(wn2bc),
             _b(pn["gamma"]), _b(pn["beta"])],
            [128, 128, 128, 128, 128], [128])

    dec = params["dec"]
    w2p = jnp.zeros((128, 128), jnp.float32).at[:, :dec["W2"].shape[1]].set(dec["W2"])
    b2p = jnp.zeros((128,), jnp.float32).at[:dec["b2"].shape[0]].set(dec["b2"])
    out = _call_rows(_dec_body, N, rb_n,
                     [h_n, dec["W1"], _b(dec["b1"]), w2p, _b(b2p)],
                     [128], [128])
    return out[:, :dec["W2"].shape[1]]


# speedup vs baseline: 1.7387x; 1.0303x over previous
import functools

import jax
import jax.numpy as jnp
from jax import lax
from jax.experimental import pallas as pl
from jax.experimental.pallas import tpu as pltpu
from jax.experimental.pallas import tpu_sc as plsc

_NC = 2
_NS = 16
_NW = _NC * _NS
_LN_EPS = 1e-5
_CHUNK = 128


def _ln_centered(d, gamma, beta):
    var = jnp.mean(d * d, axis=-1, keepdims=True)
    return d * (lax.rsqrt(var + _LN_EPS) * gamma) + beta


def _silu(x):
    return x * jax.nn.sigmoid(x)


def _enc_body(x_ref, w1_ref, b1_ref, w2_ref, b2_ref, g_ref, be_ref, o_ref):
    h = jnp.dot(x_ref[...], w1_ref[...], preferred_element_type=jnp.float32)
    h = _silu(h + b1_ref[...])
    d = jnp.dot(h, w2_ref[...], preferred_element_type=jnp.float32) + b2_ref[...]
    o_ref[...] = _ln_centered(d, g_ref[...], be_ref[...])


def _edge_body(he_ref, gadd_ref, w1_ref, b1_ref, w2_ref, b2_ref, g_ref,
               be_ref, o_ref):
    he = he_ref[...]
    h = jnp.dot(he, w1_ref[...], preferred_element_type=jnp.float32)
    h = _silu(h + gadd_ref[...] + b1_ref[...])
    d = jnp.dot(h, w2_ref[...], preferred_element_type=jnp.float32) + b2_ref[...]
    o_ref[...] = he + _ln_centered(d, g_ref[...], be_ref[...])


def _node_body(hn_ref, p0_ref, p1_ref, p2_ref, p3_ref, w1a_ref, w1b_ref,
               b1_ref, w2_ref, b2_ref, g_ref, be_ref, ws_ref, wd_ref,
               o_ref, ps_ref, pd_ref):
    hn = hn_ref[...]
    agg = (p0_ref[...] + p1_ref[...]) + (p2_ref[...] + p3_ref[...])
    h = jnp.dot(hn, w1a_ref[...], preferred_element_type=jnp.float32)
    h = h + jnp.dot(agg, w1b_ref[...], preferred_element_type=jnp.float32)
    h = _silu(h + b1_ref[...])
    d = jnp.dot(h, w2_ref[...], preferred_element_type=jnp.float32) + b2_ref[...]
    hn_new = hn + _ln_centered(d, g_ref[...], be_ref[...])
    o_ref[...] = hn_new
    ps_ref[...] = jnp.dot(hn_new, ws_ref[...], preferred_element_type=jnp.float32)
    pd_ref[...] = jnp.dot(hn_new, wd_ref[...], preferred_element_type=jnp.float32)


def _proj_body(hn_ref, ws_ref, wd_ref, ps_ref, pd_ref):
    hn = hn_ref[...]
    ps_ref[...] = jnp.dot(hn, ws_ref[...], preferred_element_type=jnp.float32)
    pd_ref[...] = jnp.dot(hn, wd_ref[...], preferred_element_type=jnp.float32)


def _dec_body(hn_ref, w1_ref, b1_ref, w2_ref, b2_ref, o_ref):
    h = jnp.dot(hn_ref[...], w1_ref[...], preferred_element_type=jnp.float32)
    h = _silu(h + b1_ref[...])
    o_ref[...] = jnp.dot(h, w2_ref[...], preferred_element_type=jnp.float32) + b2_ref[...]


def _row_spec(rb, d):
    return pl.BlockSpec((rb, d), lambda i: (i, 0))


def _full_spec(shape):
    return pl.BlockSpec(shape, lambda i: tuple(0 for _ in shape))


def _call_rows(body, rows, rb, ins, in_dims, out_dims):
    n_row = len(in_dims)
    specs = [_row_spec(rb, d) for d in in_dims]
    specs += [_full_spec(x.shape) for x in ins[n_row:]]
    multi = len(out_dims) > 1
    out_shape = [jax.ShapeDtypeStruct((rows, d), jnp.float32) for d in out_dims]
    out_spec = [_row_spec(rb, d) for d in out_dims]
    if not multi:
        out_shape, out_spec = out_shape[0], out_spec[0]
    return pl.pallas_call(
        body,
        grid=(rows // rb,),
        in_specs=specs,
        out_specs=out_spec,
        out_shape=out_shape,
    )(*ins)


def _sc_mesh():
    return plsc.VectorSubcoreMesh(core_axis_name="c", subcore_axis_name="s",
                                  num_cores=_NC, num_subcores=_NS)


def _make_gather(E):
    n_chunks = E // _CHUNK
    n_even = (n_chunks // _NW) * _NW
    per_tile = n_even // _NW
    n_k = per_tile // 3

    @functools.partial(
        pl.kernel,
        out_type=jax.ShapeDtypeStruct((E, 128), jnp.float32),
        mesh=_sc_mesh(),
        scratch_types=(
            [pltpu.VMEM((2, _CHUNK), jnp.int32)] * 3
            + [pltpu.VMEM((_CHUNK, 128), jnp.float32)] * 6
            + [pltpu.SemaphoreType.DMA] * 9
        ),
    )
    def gather_k(ps_hbm, pd_hbm, sd2_hbm, g_hbm,
                 ix0, ix1, ix2, ba0, ba1, ba2, bb0, bb1, bb2,
                 ls0, ls1, ls2, gs0, gs1, gs2, ws0, ws1, ws2):
        idx_v = [ix0, ix1, ix2]
        buf_a, buf_b = [ba0, ba1, ba2], [bb0, bb1, bb2]
        lsem, gsem, wsem = [ls0, ls1, ls2], [gs0, gs1, gs2], [ws0, ws1, ws2]
        wid = lax.axis_index("c") * _NS + lax.axis_index("s")

        def fire_idx(t, s):
            pltpu.async_copy(sd2_hbm.at[wid + t * _NW], idx_v[s], lsem[s])

        def wait_idx(s):
            pltpu.make_async_copy(sd2_hbm.at[0], idx_v[s], lsem[s]).wait()

        def fire_gather(s):
            pltpu.async_copy(ps_hbm.at[idx_v[s].at[0]], buf_a[s], gsem[s])
            pltpu.async_copy(pd_hbm.at[idx_v[s].at[1]], buf_b[s], gsem[s])

        def wait_gather(s):
            pltpu.make_async_copy(ps_hbm.at[idx_v[s].at[0]], buf_a[s], gsem[s]).wait()
            pltpu.make_async_copy(pd_hbm.at[idx_v[s].at[1]], buf_b[s], gsem[s]).wait()

        def add_slot(s):
            def body(i, _):
                for kk in range(8):
                    sl = pl.ds(kk * 16, 16)
                    buf_a[s][i, sl] = buf_a[s][i, sl] + buf_b[s][i, sl]
                return 0
            lax.fori_loop(0, _CHUNK, body, 0)

        def fire_wo(t, s):
            base = (wid + t * _NW) * _CHUNK
            pltpu.async_copy(buf_a[s], g_hbm.at[pl.ds(base, _CHUNK)], wsem[s])

        def wait_wo(s):
            pltpu.make_async_copy(buf_a[s], g_hbm.at[pl.ds(0, _CHUNK)], wsem[s]).wait()

        fire_idx(0, 0)
        wait_idx(0); fire_gather(0); fire_idx(1, 1)
        wait_idx(1); fire_gather(1); fire_idx(2, 2)
        wait_idx(2); fire_gather(2)
        wait_gather(0); add_slot(0); fire_wo(0, 0); fire_idx(3, 0)

        def loop_body(k, _):
            for s in range(3):
                t = 3 * k + s
                s2 = (s + 1) % 3
                wait_idx(s)
                wait_wo(s)
                fire_gather(s)
                wait_gather(s2)
                add_slot(s2)
                fire_wo(t - 2, s2)

                @pl.when(t + 1 < per_tile)
                def _():
                    fire_idx(t + 1, s2)
            return 0

        lax.fori_loop(1, n_k, loop_body, 0)

        wait_gather(1); add_slot(1); fire_wo(per_tile - 2, 1)
        wait_gather(2); add_slot(2); fire_wo(per_tile - 1, 2)
        wait_wo(0); wait_wo(1); wait_wo(2)

        @pl.when(wid < n_chunks - n_even)
        def _():
            c = n_even + wid
            pltpu.sync_copy(sd2_hbm.at[c], idx_v[0])
            fire_gather(0)
            wait_gather(0)
            add_slot(0)
            pltpu.sync_copy(buf_a[0], g_hbm.at[pl.ds(c * _CHUNK, _CHUNK)])

    return gather_k


def _make_scatter(E, N_pad):
    n_chunks = E // _CHUNK
    n_even = (n_chunks // _NW) * _NW
    per_tile = n_even // _NW
    n_k = per_tile // 3
    rows_per_tile = N_pad // _NS

    @functools.partial(
        pl.kernel,
        out_type=jax.ShapeDtypeStruct((_NC, N_pad, 128), jnp.float32),
        mesh=_sc_mesh(),
        scratch_types=(
            [pltpu.VMEM((_CHUNK,), jnp.int32)] * 3
            + [pltpu.VMEM((_CHUNK, 128), jnp.float32)] * 3
            + [pltpu.VMEM_SHARED((N_pad, 128), jnp.float32)]
            + [pltpu.SemaphoreType.DMA] * 6
        ),
    )
    def scatter_k(he_hbm, dst_hbm, parts_hbm,
                  ix0, ix1, ix2, rw0, rw1, rw2, acc,
                  ls0, ls1, ls2, ss0, ss1, ss2):
        idx_v, rows_v = [ix0, ix1, ix2], [rw0, rw1, rw2]
        lsem, ssem = [ls0, ls1, ls2], [ss0, ss1, ss2]
        cid = lax.axis_index("c")
        sid = lax.axis_index("s")
        wid = cid * _NS + sid
        row0 = sid * rows_per_tile

        def fire_load(t, s):
            base = (wid + t * _NW) * _CHUNK
            pltpu.async_copy(dst_hbm.at[pl.ds(base, _CHUNK)], idx_v[s], lsem[s])
            pltpu.async_copy(he_hbm.at[pl.ds(base, _CHUNK)], rows_v[s], lsem[s])

        def wait_load(s):
            pltpu.make_async_copy(dst_hbm.at[pl.ds(0, _CHUNK)], idx_v[s], lsem[s]).wait()
            pltpu.make_async_copy(he_hbm.at[pl.ds(0, _CHUNK)], rows_v[s], lsem[s]).wait()

        def fire_scatter(s):
            pltpu.async_copy(rows_v[s], acc.at[idx_v[s]], ssem[s], add=True)

        def wait_scatter(s):
            pltpu.make_async_copy(rows_v[s], acc.at[idx_v[s]], ssem[s]).wait()

        fire_load(0, 0)
        zbuf = rows_v[2]

        def zero_body(i, _):
            for kk in range(8):
                zbuf[i, pl.ds(kk * 16, 16)] = jnp.zeros((16,), jnp.float32)
            return 0

        lax.fori_loop(0, _CHUNK, zero_body, 0)
        n_full = rows_per_tile // _CHUNK
        for j in range(n_full):
            pltpu.sync_copy(zbuf, acc.at[pl.ds(row0 + j * _CHUNK, _CHUNK)])
        rem = rows_per_tile - n_full * _CHUNK
        if rem:
            pltpu.sync_copy(zbuf.at[pl.ds(0, rem)],
                            acc.at[pl.ds(row0 + n_full * _CHUNK, rem)])
        plsc.subcore_barrier()

        wait_load(0); fire_load(1, 1); fire_scatter(0)
        wait_load(1); fire_load(2, 2); fire_scatter(1)
        wait_load(2); wait_scatter(0); fire_load(3, 0); fire_scatter(2)

        def loop_body(k, _):
            for s in range(3):
                t = 3 * k + s
                s2 = (s + 1) % 3
                wait_load(s)
                wait_scatter(s2)

                @pl.when(t + 1 < per_tile)
                def _():
                    fire_load(t + 1, s2)

                fire_scatter(s)
            return 0

        lax.fori_loop(1, n_k, loop_body, 0)
        wait_scatter(1)
        wait_scatter(2)

        @pl.when(wid < n_chunks - n_even)
        def _():
            base = (n_even + wid) * _CHUNK
            pltpu.sync_copy(dst_hbm.at[pl.ds(base, _CHUNK)], idx_v[0])
            pltpu.sync_copy(he_hbm.at[pl.ds(base, _CHUNK)], rows_v[0])
            pltpu.sync_copy(rows_v[0], acc.at[idx_v[0]], add=True)

        plsc.subcore_barrier()
        pltpu.sync_copy(acc.at[pl.ds(row0, rows_per_tile)],
                        parts_hbm.at[cid, pl.ds(row0, rows_per_tile)])

    return scatter_k


def _b(v):
    return v.reshape(1, -1)


def kernel(node_feats, edge_index, edge_feats, params):
    N, node_in = node_feats.shape
    E, edge_in = edge_feats.shape
    src = edge_index[0]
    dst = edge_index[1]

    rb_n = 1000
    rb_e = 2000

    def center(p):
        w2 = p["W2"] - jnp.mean(p["W2"], axis=1, keepdims=True)
        b2 = p["b2"] - jnp.mean(p["b2"])
        return w2, b2

    def enc(x, p, rb, rows, din):
        w2c, b2c = center(p)
        return _call_rows(
            _enc_body, rows, rb,
            [x, p["W1"], _b(p["b1"]), w2c, _b(b2c),
             _b(p["gamma"]), _b(p["beta"])],
            [din], [128])

    h_n = enc(node_feats, params["node_enc"], rb_n, N, node_in)
    E2 = E // 2
    n_pad = ((N + 8 * _NS - 1) // (8 * _NS)) * (8 * _NS)
    dst_h = [dst[:E2], dst[E2:]]

    def mk_sd2(s_half, d_half):
        return jnp.concatenate(
            [s_half.reshape(-1, 1, _CHUNK),
             d_half.reshape(-1, 1, _CHUNK)], axis=1)

    sd2_h = [mk_sd2(src[:E2], dst[:E2]), mk_sd2(src[E2:], dst[E2:])]
    h_e = [enc(edge_feats[:E2], params["edge_enc"], rb_e, E2, edge_in),
           enc(edge_feats[E2:], params["edge_enc"], rb_e, E2, edge_in)]

    gather_k = _make_gather(E2)
    scatter_k = _make_scatter(E2, n_pad)

    def w1_split(i):
        w1 = params["edge_mlps"][i]["W1"]
        return w1[0:128], w1[128:256], w1[256:384]

    w1s0, w1d0, _ = w1_split(0)
    ps, pd = _call_rows(_proj_body, N, rb_n, [h_n, w1s0, w1d0],
                        [128], [128, 128])

    for i in range(6):
        pe = params["edge_mlps"][i]
        pn = params["node_mlps"][i]
        w1e = w1_split(i)[2]
        ws_next, wd_next, _ = w1_split(min(i + 1, 5))

        we2c, we2bc = center(pe)
        wn2c, wn2bc = center(pn)

        parts = [None, None]
        for h in range(2):
            g = gather_k(ps, pd, sd2_h[h])
            h_e[h] = _call_rows(
                _edge_body, E2, rb_e,
                [h_e[h], g, w1e, _b(pe["b1"]), we2c, _b(we2bc),
                 _b(pe["gamma"]), _b(pe["beta"])],
                [128, 128], [128])
            parts[h] = scatter_k(h_e[h], dst_h[h])
        h_n, ps, pd = _call_rows(
            _node_body, N, rb_n,
            [h_n, parts[0][0, :N], parts[0][1, :N], parts[1][0, :N],
             parts[1][1, :N], pn["W1"][0:128], pn["W1"][128:256],
             _b(pn["b1"]), wn2c, _b(wn2bc),
             _b(pn["gamma"]), _b(pn["beta"]), ws_next, wd_next],
            [128, 128, 128, 128, 128], [128, 128, 128])

    dec = params["dec"]
    w2p = jnp.zeros((128, 128), jnp.float32).at[:, :dec["W2"].shape[1]].set(dec["W2"])
    b2p = jnp.zeros((128,), jnp.float32).at[:dec["b2"].shape[0]].set(dec["b2"])
    out = _call_rows(_dec_body, N, rb_n,
                     [h_n, dec["W1"], _b(dec["b1"]), w2p, _b(b2p)],
                     [128], [128])
    return out[:, :dec["W2"].shape[1]]
